# baseline (speedup 1.0000x reference)
import functools

import jax
import jax.numpy as jnp
from jax import lax
from jax.experimental import pallas as pl
from jax.experimental.pallas import tpu as pltpu
from jax.experimental.pallas import tpu_sc as plsc

NC, NS = 2, 16
NW = NC * NS
B = 16384
C = 128
S = B // NW
NCH = S // C
ED = 64
PD = 32


def _sc_gather_all(idx2, tables):
    mesh = plsc.VectorSubcoreMesh(
        core_axis_name="c", subcore_axis_name="s",
        num_cores=NC, num_subcores=NS)
    dims = (ED, ED, ED, PD, PD, PD, PD, PD, PD)
    tsel = (0, 1, 1, 2, 2, 3, 3, 4, 4)

    out_type = [jax.ShapeDtypeStruct((B, d), jnp.float32) for d in dims]
    scratch = (
        [pltpu.VMEM((NCH, C), jnp.int32) for _ in range(9)]
        + [pltpu.VMEM((C, d), jnp.float32) for d in dims]
        + [pltpu.SemaphoreType.DMA]
    )

    @functools.partial(
        pl.kernel, mesh=mesh, out_type=out_type, scratch_types=scratch,
        name="bpr_sc_gather",
        compiler_params=pltpu.CompilerParams(use_tc_tiling_on_sc=False),
    )
    def k(*refs):
        idx_hbm = refs[0:9]
        tab_hbm = refs[9:14]
        out_hbm = refs[14:23]
        idx_v = refs[23:32]
        row_v = refs[32:41]
        sem = refs[41]

        wid = lax.axis_index("s") * NC + lax.axis_index("c")
        row0 = wid * NCH

        for t in range(9):
            pltpu.sync_copy(idx_hbm[t].at[pl.ds(row0, NCH)], idx_v[t])

        def chunk(g):
            off = pl.multiple_of((row0 + g) * C, C)
            handles = [
                pltpu.async_copy(
                    tab_hbm[tsel[t]].at[idx_v[t].at[g]], row_v[t], sem)
                for t in range(9)
            ]
            for h in handles:
                h.wait()
            for t in range(9):
                pltpu.sync_copy(row_v[t], out_hbm[t].at[pl.ds(off, C)])

        pl.loop(0, NCH)(chunk)

    return k(*idx2, *tables)


_BM = 2048


def _tc_body(u_ref, pi_ref, ni_ref, pc_ref, nc_ref, ppt_ref, npt_ref,
             pv_ref, nv_ref, w_ref, b_ref, o_ref):
    dot = functools.partial(
        lax.dot, precision=lax.Precision.HIGHEST,
        preferred_element_type=jnp.float32)
    w1 = w_ref[0:ED, :]
    w2 = w_ref[ED:ED + PD, :]
    w3 = w_ref[ED + PD:ED + 2 * PD, :]
    w4 = w_ref[ED + 2 * PD:ED + 3 * PD, :]
    bb = b_ref[0:1, :]
    pre_p = (dot(pi_ref[...], w1) + dot(pc_ref[...], w2)
             + dot(ppt_ref[...], w3) + dot(pv_ref[...], w4) + bb)
    pre_n = (dot(ni_ref[...], w1) + dot(nc_ref[...], w2)
             + dot(npt_ref[...], w3) + dot(nv_ref[...], w4) + bb)
    u = u_ref[...]
    s = (jnp.sum(u * jnp.maximum(pre_p, 0.0), axis=1)
         - jnp.sum(u * jnp.maximum(pre_n, 0.0), axis=1))
    o_ref[...] = s[None, None, :]


def _tc_score(rows, W, b):
    grid = B // _BM
    in_specs = [
        pl.BlockSpec((_BM, r.shape[1]), lambda i: (i, 0)) for r in rows
    ] + [
        pl.BlockSpec((ED + 3 * PD, ED), lambda i: (0, 0)),
        pl.BlockSpec((1, ED), lambda i: (0, 0)),
    ]
    out = pl.pallas_call(
        _tc_body,
        grid=(grid,),
        in_specs=in_specs,
        out_specs=pl.BlockSpec((1, 1, _BM), lambda i: (i, 0, 0)),
        out_shape=jax.ShapeDtypeStruct((grid, 1, _BM), jnp.float32),
    )(*rows, W, b.reshape(1, ED))
    return out.reshape(B)


def kernel(user_ids, pos_item_ids, neg_item_ids, pos_cat, neg_cat,
           pos_prop_type, pos_prop_value, neg_prop_type, neg_prop_value,
           user_table, item_table, cat_table, ptype_table, pval_table, W, b):
    idx = [user_ids, pos_item_ids, neg_item_ids, pos_cat, neg_cat,
           pos_prop_type, neg_prop_type, pos_prop_value, neg_prop_value]
    idx2 = [i.astype(jnp.int32).reshape(B // C, C) for i in idx]
    tables = [user_table, item_table, cat_table, ptype_table, pval_table]
    rows = _sc_gather_all(idx2, tables)
    return _tc_score(rows, W, b)
